# Initial kernel scaffold; baseline (speedup 1.0000x reference)
import jax
import jax.numpy as jnp
from jax.experimental import pallas as pl


def kernel(Z, W):
    raise NotImplementedError("write your pallas kernel here")


# speedup vs baseline: 4.9898x; 4.9898x over previous
import functools

import jax
import jax.numpy as jnp
from jax import lax
from jax.experimental import pallas as pl
from jax.experimental.pallas import tpu as pltpu
from jax.experimental.pallas import tpu_sc as plsc

EMB = 64
B_TOTAL = 16384 * 200

_info = plsc.get_sparse_core_info()
_NC, _NS = _info.num_cores, _info.num_subcores
_NW = _NC * _NS
B_PER_W = B_TOTAL // _NW
CHUNK = 1024
N_CHUNKS = B_PER_W // CHUNK


def _make_gather(vocab):
  mesh = plsc.VectorSubcoreMesh(core_axis_name="c", subcore_axis_name="s")

  @functools.partial(
      pl.kernel,
      mesh=mesh,
      compiler_params=pltpu.CompilerParams(use_tc_tiling_on_sc=False),
      out_type=jax.ShapeDtypeStruct((B_TOTAL, EMB), jnp.float32),
      scratch_types=[
          pltpu.VMEM((CHUNK,), jnp.int32),
          pltpu.VMEM((CHUNK, EMB), jnp.float32),
          pltpu.SemaphoreType.DMA,
      ],
  )
  def gather_kernel(z_hbm, w_hbm, out_hbm, idx_v, rows_v, sem):
    wid = lax.axis_index("s") * _NC + lax.axis_index("c")
    base = wid * B_PER_W

    def chunk_body(j, carry):
      off = base + j * CHUNK
      pltpu.sync_copy(z_hbm.at[pl.ds(off, CHUNK)], idx_v)
      pltpu.async_copy(w_hbm.at[idx_v], rows_v, sem).wait()
      pltpu.sync_copy(rows_v, out_hbm.at[pl.ds(off, CHUNK)])
      return carry

    lax.fori_loop(0, N_CHUNKS, chunk_body, 0)

  return gather_kernel


def kernel(Z, W):
  zf = Z.reshape(-1).astype(jnp.int32)
  out = _make_gather(W.shape[0])(zf, W)
  return out.reshape(Z.shape[0], Z.shape[1], EMB)


# speedup vs baseline: 5.1758x; 1.0373x over previous
import functools

import jax
import jax.numpy as jnp
from jax import lax
from jax.experimental import pallas as pl
from jax.experimental.pallas import tpu as pltpu
from jax.experimental.pallas import tpu_sc as plsc

EMB = 64
B_TOTAL = 16384 * 200

_info = plsc.get_sparse_core_info()
_NC, _NS = _info.num_cores, _info.num_subcores
_NW = _NC * _NS
B_PER_W = B_TOTAL // _NW
CHUNK = 256
NBUF = 4
N_CHUNKS = B_PER_W // CHUNK


def _make_gather():
  mesh = plsc.VectorSubcoreMesh(core_axis_name="c", subcore_axis_name="s")

  scratch = []
  for _ in range(NBUF):
    scratch.append(pltpu.VMEM((CHUNK,), jnp.int32))
    scratch.append(pltpu.VMEM((CHUNK, EMB), jnp.float32))
    scratch.append(pltpu.SemaphoreType.DMA)
    scratch.append(pltpu.SemaphoreType.DMA)

  @functools.partial(
      pl.kernel,
      mesh=mesh,
      compiler_params=pltpu.CompilerParams(use_tc_tiling_on_sc=False),
      out_type=jax.ShapeDtypeStruct((B_TOTAL, EMB), jnp.float32),
      scratch_types=scratch,
  )
  def gather_kernel(z_hbm, w_hbm, out_hbm, *bufs):
    idx_v = bufs[0::4]
    rows_v = bufs[1::4]
    sem_g = bufs[2::4]
    sem_s = bufs[3::4]

    wid = lax.axis_index("s") * _NC + lax.axis_index("c")
    base = wid * B_PER_W

    for b in range(NBUF):
      off = base + b * CHUNK
      pltpu.sync_copy(z_hbm.at[pl.ds(off, CHUNK)], idx_v[b])
      pltpu.async_copy(w_hbm.at[idx_v[b]], rows_v[b], sem_g[b])

    def ring_body(g, carry):
      for b in range(NBUF):
        j = g * NBUF + b
        off = base + j * CHUNK
        pltpu.make_async_copy(w_hbm.at[idx_v[b]], rows_v[b], sem_g[b]).wait()
        pltpu.async_copy(rows_v[b], out_hbm.at[pl.ds(off, CHUNK)], sem_s[b])
        jn = j + NBUF

        @pl.when(jn < N_CHUNKS)
        def _():
          offn = base + jn * CHUNK
          pltpu.sync_copy(z_hbm.at[pl.ds(offn, CHUNK)], idx_v[b])
          pltpu.make_async_copy(
              rows_v[b], out_hbm.at[pl.ds(off, CHUNK)], sem_s[b]
          ).wait()
          pltpu.async_copy(w_hbm.at[idx_v[b]], rows_v[b], sem_g[b])

      return carry

    lax.fori_loop(0, N_CHUNKS // NBUF, ring_body, 0)

    for b in range(NBUF):
      off = base + (N_CHUNKS - NBUF + b) * CHUNK
      pltpu.make_async_copy(
          rows_v[b], out_hbm.at[pl.ds(off, CHUNK)], sem_s[b]
      ).wait()

  return gather_kernel


_gather = _make_gather()


def kernel(Z, W):
  zf = Z.reshape(-1).astype(jnp.int32)
  out = _gather(zf, W)
  return out.reshape(Z.shape[0], Z.shape[1], EMB)
